# Initial kernel scaffold; baseline (speedup 1.0000x reference)
import jax
import jax.numpy as jnp
from jax.experimental import pallas as pl


def kernel(x, edge_index, edge_attr, W_key, b_key, W_query, b_query, W_value, b_value, W_edge, W_skip, bias, gamma, beta):
    raise NotImplementedError("write your pallas kernel here")


# speedup vs baseline: 1.4713x; 1.4713x over previous
import functools

import jax
import jax.numpy as jnp
from jax import lax
from jax.experimental import pallas as pl
from jax.experimental.pallas import tpu as pltpu
from jax.experimental.pallas import tpu_sc as plsc

N = 10000
E = 320000
D = 128

NC = 2
NS = 16
NW = NC * NS
EPW = E // NW
C = 80
NCHUNK = EPW // C
NROW = N // C


def _kqv_body(x_ref, wk_ref, wq_ref, wv_ref, bk_ref, bq_ref, bv_ref,
              k_out, qv_out):
    xb = x_ref[...]
    k = jnp.dot(xb, wk_ref[...], preferred_element_type=jnp.float32)
    q = jnp.dot(xb, wq_ref[...], preferred_element_type=jnp.float32)
    v = jnp.dot(xb, wv_ref[...], preferred_element_type=jnp.float32)
    k_out[...] = k + bk_ref[...]
    qv_out[:, :D] = q + bq_ref[...]
    qv_out[:, D:] = v + bv_ref[...]


def _edge_body(ea_ref, we_ref, e_out):
    e_out[...] = jnp.dot(ea_ref[...], we_ref[...],
                         preferred_element_type=jnp.float32)


def _final_body(part_ref, x_ref, ws_ref, b_ref, g_ref, beta_ref, y_out):
    xb = x_ref[...]
    out = (part_ref[0] + part_ref[1]
           + jnp.dot(xb, ws_ref[...], preferred_element_type=jnp.float32)
           + b_ref[...])
    mean = jnp.mean(out, axis=0, keepdims=True)
    cent = out - mean
    var = jnp.mean(cent * cent, axis=0, keepdims=True)
    h = cent * lax.rsqrt(var + 1e-5) * g_ref[...] + beta_ref[...]
    y_out[...] = xb + jnp.maximum(h, 0.0)


def _sc_edge_body(k_hbm, qv_hbm, e_hbm, src_hbm, dst_hbm, out_hbm,
                  srcv, dstv, kdv, qvv, ev, agg, sem1, sem2, sem3):
    cid = lax.axis_index("c")
    sid = lax.axis_index("s")
    wid = sid * NC + cid
    wbase = wid * EPW

    def _zrow(i, _):
        for j in range(D // 16):
            ev[i, pl.ds(j * 16, 16)] = jnp.zeros((16,), jnp.float32)
        return 0
    lax.fori_loop(0, C, _zrow, 0)
    for z in range((NROW + NS - 1) // NS):
        idx = sid + z * NS

        @pl.when(idx < NROW)
        def _():
            pltpu.sync_copy(ev, agg.at[pl.ds(idx * C, C)])
    plsc.subcore_barrier()

    def _row(i, _):
        for j in range(D // 16):
            sl = pl.ds(j * 16, 16)
            z = kdv[i, sl] + ev[i, sl] + qvv[i, sl]
            vv = qvv[i, pl.ds(D + j * 16, 16)]
            ev[i, sl] = vv / (1.0 + jnp.exp(-z))
        return 0

    def _chunk(c, _):
        base = wbase + c * C
        pltpu.sync_copy(src_hbm.at[pl.ds(base, C)], srcv)
        pltpu.sync_copy(dst_hbm.at[pl.ds(base, C)], dstv)
        cp1 = pltpu.async_copy(k_hbm.at[dstv], kdv, sem1)
        cp2 = pltpu.async_copy(qv_hbm.at[srcv], qvv, sem2)
        cp3 = pltpu.async_copy(e_hbm.at[pl.ds(base, C)], ev, sem3)
        cp1.wait()
        cp2.wait()
        cp3.wait()
        lax.fori_loop(0, C, _row, 0)
        pltpu.sync_copy(ev, agg.at[dstv], add=True)
        return 0

    lax.fori_loop(0, NCHUNK, _chunk, 0)
    plsc.subcore_barrier()

    for z in range((NROW + NS - 1) // NS):
        idx = sid + z * NS

        @pl.when(idx < NROW)
        def _():
            pltpu.sync_copy(agg.at[pl.ds(idx * C, C)], ev)
            pltpu.sync_copy(ev, out_hbm.at[cid, pl.ds(idx * C, C)])


@functools.cache
def _sc_edge():
    return pl.kernel(
        _sc_edge_body,
        out_type=jax.ShapeDtypeStruct((NC, N, D), jnp.float32),
        mesh=plsc.VectorSubcoreMesh(core_axis_name="c", subcore_axis_name="s",
                                    num_cores=NC, num_subcores=NS),
        scratch_types=[
            pltpu.VMEM((C,), jnp.int32),
            pltpu.VMEM((C,), jnp.int32),
            pltpu.VMEM((C, D), jnp.float32),
            pltpu.VMEM((C, 2 * D), jnp.float32),
            pltpu.VMEM((C, D), jnp.float32),
            pltpu.VMEM_SHARED((N, D), jnp.float32),
            pltpu.SemaphoreType.DMA,
            pltpu.SemaphoreType.DMA,
            pltpu.SemaphoreType.DMA,
        ],
    )


def kernel(x, edge_index, edge_attr, W_key, b_key, W_query, b_query,
           W_value, b_value, W_edge, W_skip, bias, gamma, beta):
    src = edge_index[0]
    dst = edge_index[1]
    bk = b_key.reshape(1, D)
    bq = b_query.reshape(1, D)
    bv = b_value.reshape(1, D)

    RB = 1000
    kt, qvt = pl.pallas_call(
        _kqv_body,
        grid=(N // RB,),
        in_specs=[
            pl.BlockSpec((RB, D), lambda i: (i, 0)),
            pl.BlockSpec((D, D), lambda i: (0, 0)),
            pl.BlockSpec((D, D), lambda i: (0, 0)),
            pl.BlockSpec((D, D), lambda i: (0, 0)),
            pl.BlockSpec((1, D), lambda i: (0, 0)),
            pl.BlockSpec((1, D), lambda i: (0, 0)),
            pl.BlockSpec((1, D), lambda i: (0, 0)),
        ],
        out_specs=[
            pl.BlockSpec((RB, D), lambda i: (i, 0)),
            pl.BlockSpec((RB, 2 * D), lambda i: (i, 0)),
        ],
        out_shape=[
            jax.ShapeDtypeStruct((N, D), jnp.float32),
            jax.ShapeDtypeStruct((N, 2 * D), jnp.float32),
        ],
    )(x, W_key, W_query, W_value, bk, bq, bv)

    EB = 6400
    e = pl.pallas_call(
        _edge_body,
        grid=(E // EB,),
        in_specs=[
            pl.BlockSpec((EB, D), lambda i: (i, 0)),
            pl.BlockSpec((D, D), lambda i: (0, 0)),
        ],
        out_specs=pl.BlockSpec((EB, D), lambda i: (i, 0)),
        out_shape=jax.ShapeDtypeStruct((E, D), jnp.float32),
    )(edge_attr, W_edge)

    part = _sc_edge()(kt, qvt, e, src, dst)

    y = pl.pallas_call(
        _final_body,
        in_specs=[
            pl.BlockSpec((NC, N, D), lambda: (0, 0, 0)),
            pl.BlockSpec((N, D), lambda: (0, 0)),
            pl.BlockSpec((D, D), lambda: (0, 0)),
            pl.BlockSpec((1, D), lambda: (0, 0)),
            pl.BlockSpec((1, D), lambda: (0, 0)),
            pl.BlockSpec((1, D), lambda: (0, 0)),
        ],
        out_specs=pl.BlockSpec((N, D), lambda: (0, 0)),
        out_shape=jax.ShapeDtypeStruct((N, D), jnp.float32),
    )(part, x, W_skip, bias.reshape(1, D), gamma.reshape(1, D),
      beta.reshape(1, D))
    return y
